# Initial kernel scaffold; baseline (speedup 1.0000x reference)
import jax
import jax.numpy as jnp
from jax.experimental import pallas as pl


def kernel(x, edge_index):
    raise NotImplementedError("write your pallas kernel here")


# speedup vs baseline: 17.2998x; 17.2998x over previous
import functools

import jax

jax.config.update("jax_enable_x64", True)
import jax.numpy as jnp
import numpy as np
from jax import lax
from jax.experimental import pallas as pl
from jax.experimental.pallas import tpu as pltpu
from jax.experimental.pallas import tpu_sc as plsc

K = 4
ALPHA = 0.1
BETA = 0.9
N = 10000
E = 320000
D = 128

NC, NS = 2, 16
NP = 10240
C = 128
CPT = 80
EW = C * CPT
EPAD = EW * NC * NS
RPT = NP // NS

_mesh = plsc.VectorSubcoreMesh(core_axis_name="c", subcore_axis_name="s")


@functools.partial(
    pl.kernel,
    out_type=jax.ShapeDtypeStruct((NC * NP, D), jnp.float32),
    mesh=_mesh,
    scratch_types=[
        pltpu.VMEM((2, C), jnp.int32),
        pltpu.VMEM((C, D), jnp.float32),
        pltpu.VMEM_SHARED((NP, D), jnp.float32),
        pltpu.SemaphoreType.DMA,
    ],
)
def _scatter_sum(u_hbm, ep_hbm, zeros_hbm, out_hbm, idx2, rows, acc, sem):
    c = lax.axis_index("c")
    s = lax.axis_index("s")
    w = c * NS + s
    pltpu.sync_copy(zeros_hbm, acc.at[pl.ds(s * RPT, RPT)])
    plsc.subcore_barrier()

    def chunk(i, carry):
        pltpu.sync_copy(ep_hbm.at[pl.ds((w * CPT + i) * 2, 2)], idx2)
        pltpu.async_copy(u_hbm.at[idx2.at[jnp.int32(0)]], rows, sem).wait()
        pltpu.sync_copy(rows, acc.at[idx2.at[jnp.int32(1)]], add=True)
        return carry

    lax.fori_loop(jnp.int32(0), jnp.int32(CPT), chunk, jnp.int32(0))
    plsc.subcore_barrier()
    pltpu.sync_copy(acc.at[pl.ds(s * RPT, RPT)],
                    out_hbm.at[pl.ds(c * NP + s * RPT, RPT)])


_BR = 1024


def _ew_call(body, n_in):
    zero = np.int32(0)
    specs = [pl.BlockSpec((_BR, D), lambda i: (i, zero)) for _ in range(n_in)]
    return pl.pallas_call(
        body,
        grid=(NP // _BR,),
        in_specs=specs,
        out_specs=pl.BlockSpec((_BR, D), lambda i: (i, np.int32(0))),
        out_shape=jax.ShapeDtypeStruct((NP, D), jnp.float32),
    )


def _norm_body(d0, d1, out):
    out[...] = lax.rsqrt(d0[...] + d1[...])


def _setup_norm(d0, d1):
    return _ew_call(_norm_body, 2)(d0, d1)


def _combine(ca, cb, scale, p0, p1, base):
    ca = float(ca)
    cb = float(cb)

    def body(sc, a0, a1, b, out):
        out[...] = ca * sc[...] * (a0[...] + a1[...]) + cb * b[...]

    return _ew_call(body, 4)(scale, p0, p1, base)


def _mul(a, b):
    def body(x, y, out):
        out[...] = x[...] * y[...]

    return _ew_call(body, 2)(a, b)


def kernel(x, edge_index):
    x = x.astype(jnp.float32)
    ei = edge_index.astype(jnp.int32)
    pad = jnp.full((1, EPAD - E), NP - 1, jnp.int32)
    ep = jnp.concatenate([ei, jnp.broadcast_to(pad, (2, EPAD - E))], axis=1)
    ep = ep.reshape(2, EPAD // C, C).transpose(1, 0, 2).reshape(-1, C)
    x_pad = jnp.zeros((NP, D), jnp.float32).at[:N, :].set(x)
    ones = jnp.ones((NP, D), jnp.float32)
    zeros_blk = jnp.zeros((RPT, D), jnp.float32)

    degp = _scatter_sum(ones, ep, zeros_blk)
    norm = _setup_norm(degp[:NP], degp[NP:])
    n2 = _mul(norm, norm)
    u0 = _mul(norm, x_pad)

    gamma = float(np.power(BETA, K) + ALPHA * np.sum([np.power(BETA, i) for i in range(K)]))

    u = u0
    for _ in range(K - 1):
        p = _scatter_sum(u, ep, zeros_blk)
        u = _combine(BETA, ALPHA, n2, p[:NP], p[NP:], u0)
    p = _scatter_sum(u, ep, zeros_blk)
    h = _combine(BETA / gamma, ALPHA / gamma, norm, p[:NP], p[NP:], x_pad)
    return h[:N]
